# baseline (speedup 1.0000x reference)
import functools

import jax
import jax.numpy as jnp
from jax import lax
from jax.experimental import pallas as pl
from jax.experimental.pallas import tpu as pltpu
from jax.experimental.pallas import tpu_sc as plsc

N = 10000
R = 2000
A = 1000
T = 160000
D = 250
DEPTH = 2


_RB = 400
_KB = 512
_H = 512


def _mlp3_body(x1, x2, x3, w11, w12, w13, w21, w22, w23,
               b1s, b2s, out, acc1, acc2, acc3):
    k = pl.program_id(1)
    nk = pl.num_programs(1)

    @pl.when(k == 0)
    def _init():
        acc1[...] = jnp.zeros_like(acc1)
        acc2[...] = jnp.zeros_like(acc2)
        acc3[...] = jnp.zeros_like(acc3)

    acc1[...] += jnp.dot(x1[...], w11[...], preferred_element_type=jnp.float32)
    acc2[...] += jnp.dot(x2[...], w12[...], preferred_element_type=jnp.float32)
    acc3[...] += jnp.dot(x3[...], w13[...], preferred_element_type=jnp.float32)

    @pl.when(k == nk - 1)
    def _epilogue():
        s = None
        for acc, w2, j in ((acc1, w21, 0), (acc2, w22, 1), (acc3, w23, 2)):
            h = jnp.maximum(acc[...] + b1s[j, :][None, :], 0.0)
            r = jnp.maximum(
                jnp.dot(h, w2[...], preferred_element_type=jnp.float32)
                + b2s[j, :][None, :], 0.0) + h
            s = r if s is None else s + r
        out[...] = s


def _sem_mlp3(x1, x2, x3, w1s, w2s, b1s, b2s):
    grid = (N // _RB, 4096 // _KB)
    xspec = pl.BlockSpec((_RB, _KB), lambda i, k: (i, k))
    w1spec = pl.BlockSpec((_KB, _H), lambda i, k: (k, 0))
    w2spec = pl.BlockSpec((_H, _H), lambda i, k: (0, 0))
    bspec = pl.BlockSpec((3, _H), lambda i, k: (0, 0))
    out = pl.pallas_call(
        _mlp3_body,
        grid=grid,
        in_specs=[xspec, xspec, xspec, w1spec, w1spec, w1spec,
                  w2spec, w2spec, w2spec, bspec, bspec],
        out_specs=pl.BlockSpec((_RB, _H), lambda i, k: (i, 0)),
        out_shape=jax.ShapeDtypeStruct((N, _H), jnp.float32),
        scratch_shapes=[pltpu.VMEM((_RB, _H), jnp.float32)] * 3,
        compiler_params=pltpu.CompilerParams(
            dimension_semantics=("parallel", "arbitrary")),
    )(x1, x2, x3, *w1s, *w2s, b1s, b2s)
    return out[:, :500]


_NPAD = 10240
_EC = 512
_CB = 64


def _sc_multiblock(tabb, src2d, dst2d, entries, nout):
    mesh = plsc.VectorSubcoreMesh(core_axis_name="c", subcore_axis_name="s")

    @functools.partial(
        pl.kernel,
        out_type=jax.ShapeDtypeStruct((nout * _NPAD, _CB), jnp.float32),
        mesh=mesh,
        scratch_types=[
            pltpu.VMEM((4, 128), jnp.int32),
            pltpu.VMEM((4, 128), jnp.int32),
            pltpu.VMEM((_EC, _CB), jnp.float32),
            pltpu.VMEM((160, _CB), jnp.float32),
            pltpu.VMEM_SHARED((_NPAD, _CB), jnp.float32),
            pltpu.SemaphoreType.DMA,
        ],
        compiler_params=pltpu.CompilerParams(use_tc_tiling_on_sc=False),
    )
    def k(tab_hbm, src_hbm, dst_hbm, g_hbm, idx_a, idx_d, rows,
          zbuf, acc, sem):
        cid = lax.axis_index("c")
        sid = lax.axis_index("s")

        def zrow(i, _):
            for j in range(_CB // 16):
                zbuf[i, pl.ds(j * 16, 16)] = jnp.zeros((16,), jnp.float32)
            return 0
        lax.fori_loop(0, 160, zrow, 0)

        def basis(i, _):
            rows[i, pl.ds(0, 16)] = jnp.where(
                lax.iota(jnp.int32, 16) == 0, 1.0, 0.0)
            for j in range(1, _CB // 16):
                rows[i, pl.ds(j * 16, 16)] = jnp.zeros((16,), jnp.float32)
            return 0

        def edge_pass(tab_base, erow_base, cpt, out_blk, gather):
            if not gather:
                lax.fori_loop(0, _EC, basis, 0)
            for r in range(4):
                pltpu.sync_copy(zbuf,
                                acc.at[pl.ds(sid * 640 + r * 160, 160), :])
            plsc.subcore_barrier()

            def per_chunk(kk, _):
                rbase = erow_base + (sid * cpt + kk) * (_EC // 128)
                pltpu.sync_copy(dst_hbm.at[pl.ds(rbase, 4), :], idx_d)
                if gather:
                    pltpu.sync_copy(src_hbm.at[pl.ds(rbase, 4), :], idx_a)
                    if tab_base:
                        def adj(i, _):
                            for j in range(4):
                                idx_a[j, pl.ds(i * 16, 16)] = (
                                    idx_a[j, pl.ds(i * 16, 16)] + tab_base)
                            return 0
                        lax.fori_loop(0, 8, adj, 0)
                    for j in range(4):
                        pltpu.async_copy(
                            tab_hbm.at[idx_a.at[j]],
                            rows.at[pl.ds(j * 128, 128), :], sem).wait()
                for j in range(4):
                    pltpu.sync_copy(rows.at[pl.ds(j * 128, 128), :],
                                    acc.at[idx_d.at[j]], add=True)
                return 0
            lax.fori_loop(0, cpt, per_chunk, 0)
            plsc.subcore_barrier()
            pltpu.sync_copy(
                acc.at[pl.ds(sid * 640, 640), :],
                g_hbm.at[pl.ds(out_blk * _NPAD + sid * 640, 640), :])
            plsc.subcore_barrier()

        for i, (tb, eb, cpt, ob, ga) in enumerate(entries):
            pl.when(cid == i % 2)(
                functools.partial(edge_pass, tb, eb, cpt, ob, ga))

    return k(tabb, src2d, dst2d)


def _pad_edges(src, dst, dump, multiple=16 * _EC):
    t = src.shape[0]
    tp = ((t + multiple - 1) // multiple) * multiple
    pad = tp - t
    src = jnp.concatenate([src, jnp.zeros((pad,), jnp.int32)])
    dst = jnp.concatenate([dst, jnp.full((pad,), dump, jnp.int32)])
    return src, dst


def _seg_softmax(v, seg, num):
    m = jax.ops.segment_max(v, seg, num_segments=num)
    m = jnp.where(jnp.isfinite(m), m, 0.0)
    e = jnp.exp(v - m[seg])
    s = jax.ops.segment_sum(e, seg, num_segments=num)
    return e / (s[seg] + 1e-9)


def _avg3(ent_mat, rel_mat, att_mat, ent_emb, rel_emb, att_emb):
    def padt(t):
        return jnp.pad(t, ((0, _NPAD - t.shape[0]), (0, 256 - t.shape[1])))

    tabp = jnp.stack([padt(ent_emb), padt(rel_emb), padt(att_emb)])
    tabb = tabp.reshape(3, _NPAD, 4, _CB).transpose(0, 2, 1, 3)
    tabb = tabb.reshape(12 * _NPAD, _CB)

    srcs, dsts, ebase, cpts = [], [], [], []
    rb = 0
    for mat in (ent_mat, rel_mat, att_mat):
        c, d = _pad_edges(mat[1], mat[0], N)
        srcs.append(c)
        dsts.append(d)
        ebase.append(rb)
        rb += c.shape[0] // 128
        cpts.append(c.shape[0] // (16 * _EC))
    src2d = jnp.concatenate(srcs).reshape(-1, 128)
    dst2d = jnp.concatenate(dsts).reshape(-1, 128)

    entries = [((j * 4 + q) * _NPAD, ebase[j], cpts[j], j * 4 + q, True)
               for j in range(3) for q in range(4)]
    entries += [(0, ebase[j], cpts[j], 12 + j, False) for j in range(3)]
    r = _sc_multiblock(tabb, src2d, dst2d, entries, 15)
    r = r.reshape(15, _NPAD, _CB)
    s = r[:12].reshape(3, 4, _NPAD, _CB).transpose(2, 0, 1, 3)
    s = s.reshape(_NPAD, 3, 256)[:N, :, :250]
    cnt = r[12:15, :N, 0]
    return s / (cnt.T[:, :, None] + 1e-9)


def _encoders_fused(H3, rtab, adj, rid, r_val, high_adj, atts, biases):
    src, dst = adj[0], adj[1]
    rn = rtab[rid]
    att_l = [jnp.stack([a[l] for a in atts]) for l in range(DEPTH)]
    bias_l = [jnp.stack([b[l] for b in biases]) for l in range(DEPTH)]
    outs = []
    for l in range(DEPTH):
        Hs = H3[src]
        dj = jnp.einsum('tjc,tc->tj', Hs, rn)
        msg = Hs - 2.0 * dj[..., None] * rn[:, None, :]
        sc = (jax.nn.leaky_relu(jnp.einsum('tjc,jc->tj', msg, att_l[l]))
              + r_val[:, None])
        e = jnp.exp(sc)
        sm = jax.ops.segment_sum(e, dst, num_segments=N)
        alpha = e / (sm[dst] + 1e-9)
        agg = jax.ops.segment_sum(
            (alpha[..., None] * msg).reshape(T, 750), dst, num_segments=N)
        H3 = jnp.tanh(agg.reshape(N, 3, 250) + bias_l[l][None])
        outs.append(H3)

    OUT = jnp.stack(outs, axis=2).reshape(N, 3 * DEPTH * 250)
    nbl = 24
    OUTb = jnp.pad(OUT, ((0, 0), (0, nbl * _CB - 1500)))
    OUTb = OUTb.reshape(N, nbl, _CB).transpose(1, 0, 2).reshape(nbl * N, _CB)
    hsrc, hdst = _pad_edges(high_adj[0], high_adj[1], N)
    cpt = hsrc.shape[0] // (16 * _EC)
    entries = [(b * N, 0, cpt, b, True) for b in range(nbl)]
    entries += [(0, 0, cpt, nbl, False)]
    r = _sc_multiblock(OUTb, hsrc.reshape(-1, 128), hdst.reshape(-1, 128),
                       entries, nbl + 1).reshape(nbl + 1, _NPAD, _CB)
    g = r[:nbl, :N].transpose(1, 0, 2).reshape(N, nbl * _CB)[:, :1500]
    c = r[nbl, :N, 0]
    return OUT + g / (c[:, None] + 1e-9)


def _diff_gat(x, adj, al, ar):
    src, dst = adj[0], adj[1]
    sl = x @ al
    sr = x @ ar
    score = jax.nn.leaky_relu(sl[src] + sr[dst])
    alpha = _seg_softmax(score, dst, N)
    return jnp.tanh(jax.ops.segment_sum(alpha[:, None] * x[src], dst,
                                        num_segments=N))


def _norm(x):
    return x / (jnp.linalg.norm(x, axis=-1, keepdims=True) + 1e-5)


def _align_loss(emb, pairs, temp=0.1):
    e = _norm(emb)
    l = e[pairs[:, 0]]
    r = e[pairs[:, 1]]
    logits = (l @ r.T) / temp
    lbl = jnp.arange(pairs.shape[0])
    a = jax.nn.log_softmax(logits, axis=-1)[lbl, lbl]
    b = jax.nn.log_softmax(logits.T, axis=-1)[lbl, lbl]
    return -0.5 * (jnp.mean(a) + jnp.mean(b))


def kernel(train_paris, flag, adj_matrix, r_index, r_val, rel_matrix,
           att_matrix, ent_matrix, high_adj, ill_ent, ent_semantic_emb,
           rel_semantic_emb, att_semantic_emb, ent_emb, rel_emb, att_emb,
           e_att, e_bias, r_att, r_bias, a_att, a_bias, ent_W1, ent_b1,
           ent_W2, ent_b2, rel_W1, rel_b1, rel_W2, rel_b2, att_W1, att_b1,
           att_W2, att_b2, g_al, g_ar):
    H3 = _avg3(ent_matrix, rel_matrix, att_matrix, ent_emb, rel_emb, att_emb)

    rtab = rel_emb / (jnp.linalg.norm(rel_emb, axis=-1, keepdims=True) + 1e-9)

    kg = _encoders_fused(
        H3, rtab, adj_matrix, r_index[1], r_val, high_adj,
        [e_att, r_att, a_att], [e_bias, r_bias, a_bias])

    def padw(w):
        return jnp.pad(w, ((0, 0), (0, _H - w.shape[1])))

    def padw2(w):
        return jnp.pad(w, ((0, _H - w.shape[0]), (0, _H - w.shape[1])))

    def padb(b):
        return jnp.pad(b, (0, _H - b.shape[0]))

    sem = _sem_mlp3(
        ent_semantic_emb, rel_semantic_emb, att_semantic_emb,
        [padw(ent_W1), padw(rel_W1), padw(att_W1)],
        [padw2(ent_W2), padw2(rel_W2), padw2(att_W2)],
        jnp.stack([padb(ent_b1), padb(rel_b1), padb(att_b1)]),
        jnp.stack([padb(ent_b2), padb(rel_b2), padb(att_b2)]),
    )

    fo_in = jnp.concatenate([kg, sem], axis=-1)
    fo = _diff_gat(fo_in, ent_matrix, g_al, g_ar)
    out = jnp.concatenate([kg, sem, fo], axis=-1)
    total = (_align_loss(kg, train_paris) + _align_loss(sem, train_paris)
             + _align_loss(out, train_paris))
    return total
